# baseline (speedup 1.0000x reference)
import functools

import jax
import jax.numpy as jnp
import numpy as np
from jax import lax
from jax.experimental import pallas as pl
from jax.experimental.pallas import tpu as pltpu
from jax.experimental.pallas import tpu_sc as plsc

F32 = jnp.float32
I32 = jnp.int32

S = 2048
H = 768
NSEL = 8192
KCL = 32
NH = 8
DH = 96
DHP = 128
TK = 4
TKEYS = 128
VOCAB = 50257
ROWLEN = VOCAB + 7
NEG = float(-3.0e38)

NC = 2
NS = 16
NW = NC * NS
RPW = H // NW
IPW = NSEL // NW

def _wid():
    return lax.axis_index("s") * NC + lax.axis_index("c")


def _make_sc_gather_wsel(mesh):
    return functools.partial(
        pl.kernel,
        mesh=mesh,
        out_type=(
            jax.ShapeDtypeStruct((H * NSEL,), F32),
            jax.ShapeDtypeStruct((NSEL,), F32),
        ),
        scratch_types=[
            pltpu.VMEM((ROWLEN,), F32),
            pltpu.VMEM((ROWLEN,), F32),
            pltpu.VMEM((NSEL,), I32),
            pltpu.VMEM((NSEL,), F32),
            pltpu.VMEM((NSEL,), F32),
            pltpu.SemaphoreType.DMA,
            pltpu.SemaphoreType.DMA,
            pltpu.SemaphoreType.DMA,
            pltpu.SemaphoreType.DMA,
        ],
        compiler_params=pltpu.CompilerParams(needs_layout_passes=False),
    )(_sc_gather_wsel_body)


def _sc_gather_wsel_body(w2flat, b2p, ids, wsel_out, bsel_out,
                         row0, row1, ids_v, orow0, orow1,
                         dsem0, dsem1, osem0, osem1):
    wid = _wid()
    pltpu.sync_copy(ids, ids_v)
    base = wid * RPW

    def _issue(r, buf, dsem):
        hrow = base + r
        start8 = pl.multiple_of(hrow * VOCAB // 8 * 8, 8)
        pltpu.async_copy(w2flat.at[pl.ds(start8, ROWLEN)], buf, dsem)

    def _gather_row(r, buf, obuf, osem):
        hrow = base + r
        delta = hrow * VOCAB - hrow * VOCAB // 8 * 8

        def gbody(c, carry2):
            idxs = ids_v[pl.ds(c * 16, 16)] + delta
            obuf[pl.ds(c * 16, 16)] = plsc.load_gather(buf, [idxs])
            return carry2

        lax.fori_loop(0, NSEL // 16, gbody, 0, unroll=8)
        pltpu.async_copy(obuf, wsel_out.at[pl.ds(hrow * NSEL, NSEL)], osem)

    _issue(0, row0, dsem0)
    _issue(1, row1, dsem1)

    def pair(p, carry):
        r0 = 2 * p
        for r, buf, obuf, dsem, osem in ((r0, row0, orow0, dsem0, osem0),
                                         (r0 + 1, row1, orow1, dsem1, osem1)):
            pltpu.make_async_copy(w2flat.at[pl.ds(0, ROWLEN)], buf, dsem).wait()

            @pl.when(p > 0)
            def _():
                pltpu.make_async_copy(obuf, wsel_out.at[pl.ds(0, NSEL)],
                                      osem).wait()

            _gather_row(r, buf, obuf, osem)

            @pl.when(r + 2 < RPW)
            def _():
                _issue(r + 2, buf, dsem)
        return carry

    lax.fori_loop(0, RPW // 2, pair, 0)
    pltpu.make_async_copy(orow0, wsel_out.at[pl.ds(0, NSEL)], osem0).wait()
    pltpu.make_async_copy(orow1, wsel_out.at[pl.ds(0, NSEL)], osem1).wait()

    pltpu.sync_copy(b2p, row0)

    def bbody(c, carry):
        idxs = ids_v[pl.ds(wid * IPW + c * 16, 16)]
        orow0[pl.ds(c * 16, 16)] = plsc.load_gather(row0, [idxs])
        return carry

    lax.fori_loop(0, IPW // 16, bbody, 0, unroll=4)
    pltpu.sync_copy(orow0.at[pl.ds(0, IPW)], bsel_out.at[pl.ds(wid * IPW, IPW)])


def _make_sc_gather_emb(mesh):
    return functools.partial(
        pl.kernel,
        mesh=mesh,
        out_type=(
            jax.ShapeDtypeStruct((NSEL, H), F32),
            jax.ShapeDtypeStruct((TKEYS, H), F32),
        ),
        scratch_types=[
            pltpu.VMEM((NSEL,), I32),
            pltpu.VMEM((IPW,), I32),
            pltpu.VMEM((64,), I32),
            pltpu.VMEM((64, H), F32),
            pltpu.SemaphoreType.DMA,
        ],
        compiler_params=pltpu.CompilerParams(needs_layout_passes=False),
    )(_sc_gather_emb_body)


def _sc_gather_emb_body(tok, ids, tidx, kidx, g4_out, keys_out,
                        ids_v, tidx_v, fid_v, rows_v, sem):
    wid = _wid()
    pltpu.sync_copy(ids, ids_v)
    pltpu.sync_copy(tidx.at[pl.ds(wid * IPW, IPW)], tidx_v)

    def chunk(c, carry):
        def mp(k, carry2):
            idxs = tidx_v[pl.ds(c * 64 + k * 16, 16)]
            fid_v[pl.ds(k * 16, 16)] = plsc.load_gather(ids_v, [idxs])
            return carry2

        lax.fori_loop(0, 4, mp, 0, unroll=4)
        pltpu.async_copy(tok.at[fid_v], rows_v, sem).wait()
        pltpu.sync_copy(rows_v, g4_out.at[pl.ds(wid * IPW + c * 64, 64)])
        return carry

    lax.fori_loop(0, IPW // 64, chunk, 0)

    @pl.when(wid < TKEYS // 64)
    def _():
        pltpu.sync_copy(kidx.at[pl.ds(wid * 64, 64)], tidx_v.at[pl.ds(0, 64)])

        def mp2(k, carry2):
            idxs = tidx_v[pl.ds(k * 16, 16)]
            fid_v[pl.ds(k * 16, 16)] = plsc.load_gather(ids_v, [idxs])
            return carry2

        lax.fori_loop(0, 4, mp2, 0, unroll=4)
        pltpu.async_copy(tok.at[fid_v], rows_v, sem).wait()
        pltpu.sync_copy(rows_v, keys_out.at[pl.ds(wid * 64, 64)])


def _a0_body(hs, w1, b1, w2, b2, w3, b3, g, b, sw1, sb1, q_out, h1_out):
    x = hs[...]
    h = jnp.maximum(x @ w1[...] + b1[...], 0.0)
    h = jnp.maximum(h @ w2[...] + b2[...], 0.0)
    h = h @ w3[...] + b3[...]
    mu = jnp.mean(h, axis=-1, keepdims=True)
    var = jnp.mean((h - mu) ** 2, axis=-1, keepdims=True)
    q_out[...] = (h - mu) / jnp.sqrt(var + 1e-5) * g[...] + b[...]
    h1_out[...] = jnp.maximum(x @ sw1[...] + sb1[...], 0.0)


def _a0(hs, p):
    full = lambda shape: pl.BlockSpec(shape, lambda: (0,) * len(shape))
    return pl.pallas_call(
        _a0_body,
        out_shape=(
            jax.ShapeDtypeStruct((S, H), F32),
            jax.ShapeDtypeStruct((S, H), F32),
        ),
        in_specs=[full((S, H))] + [full((H, H)), full((1, H))] * 3
        + [full((1, H)), full((1, H))] + [full((H, H)), full((1, H))],
        out_specs=(full((S, H)), full((S, H))),
    )(hs, p['lvl_W1'], p['lvl_b1'].reshape(1, H),
      p['lvl_W2'], p['lvl_b2'].reshape(1, H),
      p['lvl_W3'], p['lvl_b3'].reshape(1, H),
      p['lvl_ln_g'].reshape(1, H), p['lvl_ln_b'].reshape(1, H),
      p['sel_W1'], p['sel_b1'].reshape(1, H))


def _attn_body(q_ref, kv_ref, wq, bq, wk, bk, wv, bv, wo, bo, out_ref):
    h = pl.program_id(0)
    scale = np.float32(np.sqrt(DH))
    qh = q_ref[...] @ wq[0] + bq[0]
    kh = kv_ref[...] @ wk[0] + bk[0]
    vh = kv_ref[...] @ wv[0] + bv[0]
    s = lax.dot_general(qh, kh, (((1,), (1,)), ((), ()))) / scale
    m = jnp.max(s, axis=-1, keepdims=True)
    e = jnp.exp(s - m)
    w = e / jnp.sum(e, axis=-1, keepdims=True)
    contrib = (w @ vh) @ wo[0]

    @pl.when(h == 0)
    def _():
        out_ref[...] = contrib + bo[...]

    @pl.when(h > 0)
    def _():
        out_ref[...] += contrib


def _head_w(W):
    Wh = W.reshape(H, NH, DH).transpose(1, 0, 2)
    return jnp.pad(Wh, ((0, 0), (0, 0), (0, DHP - DH)))


def _head_b(b):
    return jnp.pad(b.reshape(NH, 1, DH), ((0, 0), (0, 0), (0, DHP - DH)))


def _head_wo(W):
    return jnp.pad(W.reshape(NH, DH, H), ((0, 0), (0, DHP - DH), (0, 0)))


def _attn(q, kv, p, pre):
    nkv = kv.shape[0]
    full = lambda shape: pl.BlockSpec(shape, lambda h: (0,) * len(shape))
    headw = lambda shape: pl.BlockSpec((1,) + shape, lambda h: (h, 0, 0))
    return pl.pallas_call(
        _attn_body,
        grid=(NH,),
        out_shape=jax.ShapeDtypeStruct((S, H), F32),
        in_specs=[full((S, H)), full((nkv, H)),
                  headw((H, DHP)), headw((1, DHP)),
                  headw((H, DHP)), headw((1, DHP)),
                  headw((H, DHP)), headw((1, DHP)),
                  headw((DHP, H)), full((1, H))],
        out_specs=full((S, H)),
    )(q, kv,
      _head_w(p[pre + 'Wq']), _head_b(p[pre + 'bq']),
      _head_w(p[pre + 'Wk']), _head_b(p[pre + 'bk']),
      _head_w(p[pre + 'Wv']), _head_b(p[pre + 'bv']),
      _head_wo(p[pre + 'Wo']), p[pre + 'bo'].reshape(1, H))


RBLK = 128
NRB = S // RBLK


def _iter_topk(v, k, width):
    cols = lax.broadcasted_iota(I32, v.shape, 1)
    lane = lax.broadcasted_iota(I32, (v.shape[0], k), 1)
    out = jnp.zeros((v.shape[0], k), I32)
    for t in range(k):
        m = jnp.max(v, axis=1, keepdims=True)
        idx = jnp.min(jnp.where(v == m, cols, width), axis=1, keepdims=True)
        out = jnp.where(lane == t, idx, out)
        if t + 1 < k:
            v = jnp.where(cols == idx, NEG, v)
    return out


NCH = NSEL // 128


def _scores_top_body(h1_ref, w_ref, b_ref, idx_out, pooled_out, key_out, s_ref):
    i = pl.program_id(0)
    s = h1_ref[...] @ w_ref[...] + b_ref[...]
    s_ref[...] = s
    colsum = jnp.sum(s, axis=0, keepdims=True) * np.float32(1.0 / S)

    @pl.when(i == 0)
    def _():
        pooled_out[...] = colsum
        key_out[...] = jnp.zeros((1, TKEYS), I32)

    @pl.when(i > 0)
    def _():
        pooled_out[...] += colsum

    lane = lax.broadcasted_iota(I32, (8, 128), 1)
    neg = jnp.full((8, 128), NEG, F32)
    big = jnp.full((8, 128), NSEL, I32)
    NSUB = 4
    ccand = lax.broadcasted_iota(I32, (8, NSUB * TK * 128), 1)

    val_rows = []
    for rg in range(RBLK // 8):
        r0, r1 = rg * 8, rg * 8 + 8

        def merge(cc, carry):
            new = []
            x4 = s_ref[r0:r1, pl.ds(pl.multiple_of(cc * (NSUB * 128),
                                                   NSUB * 128), NSUB * 128)]
            for k in range(NSUB):
                t0, t1, t2, t3 = carry[TK * k:TK * k + TK]
                x = x4[:, k * 128:(k + 1) * 128]
                n0 = jnp.maximum(t0, x)
                q0 = jnp.minimum(t0, x)
                n1 = jnp.maximum(t1, q0)
                q1 = jnp.minimum(t1, q0)
                n2 = jnp.maximum(t2, q1)
                q2 = jnp.minimum(t2, q1)
                n3 = jnp.maximum(t3, q2)
                new += [n0, n1, n2, n3]
            return tuple(new)

        planes = lax.fori_loop(0, NCH // NSUB, merge, (neg,) * (TK * NSUB))
        cand = jnp.concatenate(list(planes), axis=1)
        vals = []
        for t in range(TK):
            m = jnp.max(cand, axis=1, keepdims=True)
            vals.append(m)
            if t + 1 < TK:
                fidx = jnp.min(jnp.where(cand == m, ccand, NSUB * TK * 128),
                               axis=1, keepdims=True)
                cand = jnp.where(ccand == fidx, NEG, cand)
        val_rows.append(jnp.concatenate(vals, axis=1))

    idx_rows = []
    for rg in range(RBLK // 8):
        r0, r1 = rg * 8, rg * 8 + 8
        v4 = val_rows[rg]

        def ipass(cc, carry):
            new = []
            x4 = s_ref[r0:r1, pl.ds(pl.multiple_of(cc * (NSUB * 128),
                                                   NSUB * 128), NSUB * 128)]
            for k in range(NSUB):
                accs = carry[TK * k:TK * k + TK]
                x = x4[:, k * 128:(k + 1) * 128]
                colid = lane + (cc * NSUB + k) * 128
                new += [
                    jnp.minimum(accs[t],
                                jnp.where(x == v4[:, t:t + 1], colid, NSEL))
                    for t in range(TK)]
            return tuple(new)

        accs = lax.fori_loop(0, NCH // NSUB, ipass, (big,) * (TK * NSUB))
        idxs = []
        for t in range(TK):
            a = accs[t]
            for k in range(1, NSUB):
                a = jnp.minimum(a, accs[TK * k + t])
            idxs.append(jnp.min(a, axis=1, keepdims=True))
        idx_rows.append(jnp.concatenate(idxs, axis=1))

    idx_out[...] = jnp.concatenate(idx_rows, axis=0)

    @pl.when(i == NRB - 1)
    def _():
        key_out[...] = _iter_topk(pooled_out[...], TKEYS, NSEL)


def _scores_top(h1, wsel, bsel):
    return pl.pallas_call(
        _scores_top_body,
        grid=(NRB,),
        out_shape=(
            jax.ShapeDtypeStruct((S, TK), I32),
            jax.ShapeDtypeStruct((1, NSEL), F32),
            jax.ShapeDtypeStruct((1, TKEYS), I32),
        ),
        in_specs=[pl.BlockSpec((RBLK, H), lambda i: (i, 0)),
                  pl.BlockSpec((H, NSEL), lambda i: (0, 0)),
                  pl.BlockSpec((1, NSEL), lambda i: (0, 0))],
        out_specs=(pl.BlockSpec((RBLK, TK), lambda i: (i, 0)),
                   pl.BlockSpec((1, NSEL), lambda i: (0, 0)),
                   pl.BlockSpec((1, TKEYS), lambda i: (0, 0))),
        scratch_shapes=[pltpu.VMEM((RBLK, NSEL), F32)],
    )(h1, wsel, bsel)


DBLK = 512


def _post_body(hs, ao, fa, g4, fw1, fb1, fw2, fb2,
               ow1, ob1, ow2, ob2, ow3, ob3, ow4, ob4, lg, lb, out_ref):
    gathered = (g4[:, 0, :] + g4[:, 1, :] + g4[:, 2, :] + g4[:, 3, :]) \
        * np.float32(0.25)
    ffn_in = fa[...] + gathered + ao[...]
    t = jnp.maximum(ffn_in @ fw1[...] + fb1[...], 0.0)
    ffn = t @ fw2[...] + fb2[...]
    combined = hs[...] + ffn
    o = jnp.maximum(combined @ ow1[...] + ob1[...], 0.0)
    o = jnp.maximum(o @ ow2[...] + ob2[...], 0.0)
    o = jnp.maximum(o @ ow3[...] + ob3[...], 0.0)
    o = o @ ow4[...] + ob4[...]
    mu = jnp.mean(o, axis=-1, keepdims=True)
    var = jnp.mean((o - mu) ** 2, axis=-1, keepdims=True)
    out_ref[...] = (o - mu) / jnp.sqrt(var + 1e-5) * lg[...] + lb[...]


def _post(hs, attn_out, final_attn, g4, p):
    blk = lambda shape: pl.BlockSpec(shape, lambda i: (i,) + (0,) * (len(shape) - 1))
    wfull = lambda shape: pl.BlockSpec(shape, lambda i: (0,) * len(shape))
    return pl.pallas_call(
        _post_body,
        grid=(S // DBLK,),
        out_shape=jax.ShapeDtypeStruct((S, H), F32),
        in_specs=[blk((DBLK, H)), blk((DBLK, H)), blk((DBLK, H)),
                  blk((DBLK, TK, H)),
                  wfull((H, H)), wfull((1, H)), wfull((H, H)), wfull((1, H)),
                  wfull((H, 512)), wfull((1, 512)),
                  wfull((512, 2 * H)), wfull((1, 2 * H)),
                  wfull((2 * H, 2 * H)), wfull((1, 2 * H)),
                  wfull((2 * H, H)), wfull((1, H)),
                  wfull((1, H)), wfull((1, H))],
        out_specs=blk((DBLK, H)),
    )(hs, attn_out, final_attn, g4,
      p['ffn_W1'], p['ffn_b1'].reshape(1, H),
      p['ffn_W2'], p['ffn_b2'].reshape(1, H),
      p['out_W1'], p['out_b1'].reshape(1, 512),
      p['out_W2'], p['out_b2'].reshape(1, 2 * H),
      p['out_W3'], p['out_b3'].reshape(1, 2 * H),
      p['out_W4'], p['out_b4'].reshape(1, H),
      p['out_ln_g'].reshape(1, H), p['out_ln_b'].reshape(1, H))


@functools.cache
def _sc_kernels():
    mesh = plsc.VectorSubcoreMesh(core_axis_name="c", subcore_axis_name="s")
    return _make_sc_gather_wsel(mesh), _make_sc_gather_emb(mesh)


def kernel(hidden_states, attention_mask, cluster_embeddings,
           selected_token_ids, token_embeddings, params):
    p = params
    hs = hidden_states.reshape(S, H)
    ids = selected_token_ids
    sc_wsel, sc_emb = _sc_kernels()

    w2flat = p['sel_W2'].reshape(-1)
    b2p = jnp.pad(p['sel_b2'], (0, ROWLEN - VOCAB))
    wsel_flat, bsel = sc_wsel(w2flat, b2p, ids)
    wsel = wsel_flat.reshape(H, NSEL)

    query, h1 = _a0(hs, p)
    attn_out = _attn(query, cluster_embeddings, p, 'lvl_attn_')
    top_idx, _pooled, key_idx = _scores_top(h1, wsel, bsel.reshape(1, NSEL))

    g4, keys = sc_emb(token_embeddings, ids,
                      top_idx.reshape(-1), key_idx.reshape(-1))
    final_attn = _attn(hs, keys, p, 'fin_attn_')
    out = _post(hs, attn_out, final_attn, g4.reshape(S, TK, H), p)
    return out.reshape(1, S, H)


# speedup vs baseline: 1.0034x; 1.0034x over previous
import functools

import jax
import jax.numpy as jnp
import numpy as np
from jax import lax
from jax.experimental import pallas as pl
from jax.experimental.pallas import tpu as pltpu
from jax.experimental.pallas import tpu_sc as plsc

F32 = jnp.float32
I32 = jnp.int32

S = 2048
H = 768
NSEL = 8192
KCL = 32
NH = 8
DH = 96
DHP = 128
TK = 4
TKEYS = 128
VOCAB = 50257
ROWLEN = VOCAB + 7
NEG = float(-3.0e38)

NC = 2
NS = 16
NW = NC * NS
RPW = H // NW
IPW = NSEL // NW

def _wid():
    return lax.axis_index("s") * NC + lax.axis_index("c")


def _make_sc_gather_wsel(mesh):
    return functools.partial(
        pl.kernel,
        mesh=mesh,
        out_type=(
            jax.ShapeDtypeStruct((H * NSEL,), F32),
            jax.ShapeDtypeStruct((NSEL,), F32),
        ),
        scratch_types=[
            pltpu.VMEM((ROWLEN,), F32),
            pltpu.VMEM((ROWLEN,), F32),
            pltpu.VMEM((NSEL,), I32),
            pltpu.VMEM((NSEL,), F32),
            pltpu.VMEM((NSEL,), F32),
            pltpu.SemaphoreType.DMA,
            pltpu.SemaphoreType.DMA,
            pltpu.SemaphoreType.DMA,
            pltpu.SemaphoreType.DMA,
        ],
        compiler_params=pltpu.CompilerParams(needs_layout_passes=False),
    )(_sc_gather_wsel_body)


def _sc_gather_wsel_body(w2flat, b2p, ids, wsel_out, bsel_out,
                         row0, row1, ids_v, orow0, orow1,
                         dsem0, dsem1, osem0, osem1):
    wid = _wid()
    pltpu.sync_copy(ids, ids_v)
    base = wid * RPW

    def _issue(r, buf, dsem):
        hrow = base + r
        start8 = pl.multiple_of(hrow * VOCAB // 8 * 8, 8)
        pltpu.async_copy(w2flat.at[pl.ds(start8, ROWLEN)], buf, dsem)

    def _gather_row(r, buf, obuf, osem):
        hrow = base + r
        delta = hrow * VOCAB - hrow * VOCAB // 8 * 8

        def gbody(c, carry2):
            idxs = ids_v[pl.ds(c * 16, 16)] + delta
            obuf[pl.ds(c * 16, 16)] = plsc.load_gather(buf, [idxs])
            return carry2

        lax.fori_loop(0, NSEL // 16, gbody, 0, unroll=8)
        pltpu.async_copy(obuf, wsel_out.at[pl.ds(hrow * NSEL, NSEL)], osem)

    _issue(0, row0, dsem0)
    _issue(1, row1, dsem1)

    def pair(p, carry):
        r0 = 2 * p
        for r, buf, obuf, dsem, osem in ((r0, row0, orow0, dsem0, osem0),
                                         (r0 + 1, row1, orow1, dsem1, osem1)):
            pltpu.make_async_copy(w2flat.at[pl.ds(0, ROWLEN)], buf, dsem).wait()

            @pl.when(p > 0)
            def _():
                pltpu.make_async_copy(obuf, wsel_out.at[pl.ds(0, NSEL)],
                                      osem).wait()

            _gather_row(r, buf, obuf, osem)

            @pl.when(r + 2 < RPW)
            def _():
                _issue(r + 2, buf, dsem)
        return carry

    lax.fori_loop(0, RPW // 2, pair, 0)
    pltpu.make_async_copy(orow0, wsel_out.at[pl.ds(0, NSEL)], osem0).wait()
    pltpu.make_async_copy(orow1, wsel_out.at[pl.ds(0, NSEL)], osem1).wait()

    pltpu.sync_copy(b2p, row0)

    def bbody(c, carry):
        idxs = ids_v[pl.ds(wid * IPW + c * 16, 16)]
        orow0[pl.ds(c * 16, 16)] = plsc.load_gather(row0, [idxs])
        return carry

    lax.fori_loop(0, IPW // 16, bbody, 0, unroll=4)
    pltpu.sync_copy(orow0.at[pl.ds(0, IPW)], bsel_out.at[pl.ds(wid * IPW, IPW)])


def _make_sc_gather_emb(mesh):
    return functools.partial(
        pl.kernel,
        mesh=mesh,
        out_type=(
            jax.ShapeDtypeStruct((NSEL, H), F32),
            jax.ShapeDtypeStruct((TKEYS, H), F32),
        ),
        scratch_types=[
            pltpu.VMEM((NSEL,), I32),
            pltpu.VMEM((IPW,), I32),
            pltpu.VMEM((64,), I32),
            pltpu.VMEM((64, H), F32),
            pltpu.SemaphoreType.DMA,
        ],
        compiler_params=pltpu.CompilerParams(needs_layout_passes=False),
    )(_sc_gather_emb_body)


def _sc_gather_emb_body(tok, ids, tidx, kidx, g4_out, keys_out,
                        ids_v, tidx_v, fid_v, rows_v, sem):
    wid = _wid()
    pltpu.sync_copy(ids, ids_v)
    pltpu.sync_copy(tidx.at[pl.ds(wid * IPW, IPW)], tidx_v)

    def chunk(c, carry):
        def mp(k, carry2):
            idxs = tidx_v[pl.ds(c * 64 + k * 16, 16)]
            fid_v[pl.ds(k * 16, 16)] = plsc.load_gather(ids_v, [idxs])
            return carry2

        lax.fori_loop(0, 4, mp, 0, unroll=4)
        pltpu.async_copy(tok.at[fid_v], rows_v, sem).wait()
        pltpu.sync_copy(rows_v, g4_out.at[pl.ds(wid * IPW + c * 64, 64)])
        return carry

    lax.fori_loop(0, IPW // 64, chunk, 0)

    @pl.when(wid < TKEYS // 64)
    def _():
        pltpu.sync_copy(kidx.at[pl.ds(wid * 64, 64)], tidx_v.at[pl.ds(0, 64)])

        def mp2(k, carry2):
            idxs = tidx_v[pl.ds(k * 16, 16)]
            fid_v[pl.ds(k * 16, 16)] = plsc.load_gather(ids_v, [idxs])
            return carry2

        lax.fori_loop(0, 4, mp2, 0, unroll=4)
        pltpu.async_copy(tok.at[fid_v], rows_v, sem).wait()
        pltpu.sync_copy(rows_v, keys_out.at[pl.ds(wid * 64, 64)])


def _a0_body(hs, w1, b1, w2, b2, w3, b3, g, b, sw1, sb1, q_out, h1_out):
    x = hs[...]
    h = jnp.maximum(x @ w1[...] + b1[...], 0.0)
    h = jnp.maximum(h @ w2[...] + b2[...], 0.0)
    h = h @ w3[...] + b3[...]
    mu = jnp.mean(h, axis=-1, keepdims=True)
    var = jnp.mean((h - mu) ** 2, axis=-1, keepdims=True)
    q_out[...] = (h - mu) / jnp.sqrt(var + 1e-5) * g[...] + b[...]
    h1_out[...] = jnp.maximum(x @ sw1[...] + sb1[...], 0.0)


def _a0(hs, p):
    full = lambda shape: pl.BlockSpec(shape, lambda: (0,) * len(shape))
    return pl.pallas_call(
        _a0_body,
        out_shape=(
            jax.ShapeDtypeStruct((S, H), F32),
            jax.ShapeDtypeStruct((S, H), F32),
        ),
        in_specs=[full((S, H))] + [full((H, H)), full((1, H))] * 3
        + [full((1, H)), full((1, H))] + [full((H, H)), full((1, H))],
        out_specs=(full((S, H)), full((S, H))),
    )(hs, p['lvl_W1'], p['lvl_b1'].reshape(1, H),
      p['lvl_W2'], p['lvl_b2'].reshape(1, H),
      p['lvl_W3'], p['lvl_b3'].reshape(1, H),
      p['lvl_ln_g'].reshape(1, H), p['lvl_ln_b'].reshape(1, H),
      p['sel_W1'], p['sel_b1'].reshape(1, H))


def _attn_body(q_ref, kv_ref, wq, bq, wk, bk, wv, bv, wo, bo, out_ref):
    h = pl.program_id(0)
    scale = np.float32(np.sqrt(DH))
    qh = q_ref[...] @ wq[0] + bq[0]
    kh = kv_ref[...] @ wk[0] + bk[0]
    vh = kv_ref[...] @ wv[0] + bv[0]
    s = lax.dot_general(qh, kh, (((1,), (1,)), ((), ()))) / scale
    m = jnp.max(s, axis=-1, keepdims=True)
    e = jnp.exp(s - m)
    w = e / jnp.sum(e, axis=-1, keepdims=True)
    contrib = (w @ vh) @ wo[0]

    @pl.when(h == 0)
    def _():
        out_ref[...] = contrib + bo[...]

    @pl.when(h > 0)
    def _():
        out_ref[...] += contrib


def _head_w(W):
    Wh = W.reshape(H, NH, DH).transpose(1, 0, 2)
    return jnp.pad(Wh, ((0, 0), (0, 0), (0, DHP - DH)))


def _head_b(b):
    return jnp.pad(b.reshape(NH, 1, DH), ((0, 0), (0, 0), (0, DHP - DH)))


def _head_wo(W):
    return jnp.pad(W.reshape(NH, DH, H), ((0, 0), (0, DHP - DH), (0, 0)))


def _attn(q, kv, p, pre):
    nkv = kv.shape[0]
    full = lambda shape: pl.BlockSpec(shape, lambda h: (0,) * len(shape))
    headw = lambda shape: pl.BlockSpec((1,) + shape, lambda h: (h, 0, 0))
    return pl.pallas_call(
        _attn_body,
        grid=(NH,),
        out_shape=jax.ShapeDtypeStruct((S, H), F32),
        in_specs=[full((S, H)), full((nkv, H)),
                  headw((H, DHP)), headw((1, DHP)),
                  headw((H, DHP)), headw((1, DHP)),
                  headw((H, DHP)), headw((1, DHP)),
                  headw((DHP, H)), full((1, H))],
        out_specs=full((S, H)),
    )(q, kv,
      _head_w(p[pre + 'Wq']), _head_b(p[pre + 'bq']),
      _head_w(p[pre + 'Wk']), _head_b(p[pre + 'bk']),
      _head_w(p[pre + 'Wv']), _head_b(p[pre + 'bv']),
      _head_wo(p[pre + 'Wo']), p[pre + 'bo'].reshape(1, H))


RBLK = 128
NRB = S // RBLK


def _iter_topk(v, k, width):
    cols = lax.broadcasted_iota(I32, v.shape, 1)
    lane = lax.broadcasted_iota(I32, (v.shape[0], k), 1)
    out = jnp.zeros((v.shape[0], k), I32)
    for t in range(k):
        m = jnp.max(v, axis=1, keepdims=True)
        idx = jnp.min(jnp.where(v == m, cols, width), axis=1, keepdims=True)
        out = jnp.where(lane == t, idx, out)
        if t + 1 < k:
            v = jnp.where(cols == idx, NEG, v)
    return out


NCH = NSEL // 128
NBLK = 1024


def _scores_body(h1_ref, w_ref, b_ref, s_out, p_out):
    s = h1_ref[...] @ w_ref[...] + b_ref[...]
    s_out[...] = s
    p_out[...] = jnp.sum(s, axis=0, keepdims=True) * np.float32(1.0 / S)


def _scores(h1, wsel, bsel):
    return pl.pallas_call(
        _scores_body,
        grid=(NSEL // NBLK,),
        out_shape=(
            jax.ShapeDtypeStruct((S, NSEL), F32),
            jax.ShapeDtypeStruct((1, NSEL), F32),
        ),
        in_specs=[pl.BlockSpec((S, H), lambda j: (0, 0)),
                  pl.BlockSpec((H, NBLK), lambda j: (0, j)),
                  pl.BlockSpec((1, NBLK), lambda j: (0, j))],
        out_specs=(pl.BlockSpec((S, NBLK), lambda j: (0, j)),
                   pl.BlockSpec((1, NBLK), lambda j: (0, j))),
    )(h1, wsel, bsel)


def _topkeys(pooled):
    full = lambda shape: pl.BlockSpec(shape, lambda: (0,) * len(shape))
    return pl.pallas_call(
        lambda p_ref, idx_out: idx_out.__setitem__(
            ..., _iter_topk(p_ref[...], TKEYS, NSEL)),
        out_shape=jax.ShapeDtypeStruct((1, TKEYS), I32),
        in_specs=[full((1, NSEL))],
        out_specs=full((1, TKEYS)),
    )(pooled)


def _top4_body(s_ref, idx_out):
    lane = lax.broadcasted_iota(I32, (8, 128), 1)
    neg = jnp.full((8, 128), NEG, F32)
    big = jnp.full((8, 128), NSEL, I32)
    NSUB = 4
    ccand = lax.broadcasted_iota(I32, (8, NSUB * TK * 128), 1)

    val_rows = []
    for rg in range(RBLK // 8):
        r0, r1 = rg * 8, rg * 8 + 8

        def merge(cc, carry):
            new = []
            x4 = s_ref[r0:r1, pl.ds(pl.multiple_of(cc * (NSUB * 128),
                                                   NSUB * 128), NSUB * 128)]
            for k in range(NSUB):
                t0, t1, t2, t3 = carry[TK * k:TK * k + TK]
                x = x4[:, k * 128:(k + 1) * 128]
                n0 = jnp.maximum(t0, x)
                q0 = jnp.minimum(t0, x)
                n1 = jnp.maximum(t1, q0)
                q1 = jnp.minimum(t1, q0)
                n2 = jnp.maximum(t2, q1)
                q2 = jnp.minimum(t2, q1)
                n3 = jnp.maximum(t3, q2)
                new += [n0, n1, n2, n3]
            return tuple(new)

        planes = lax.fori_loop(0, NCH // NSUB, merge, (neg,) * (TK * NSUB))
        cand = jnp.concatenate(list(planes), axis=1)
        vals = []
        for t in range(TK):
            m = jnp.max(cand, axis=1, keepdims=True)
            vals.append(m)
            if t + 1 < TK:
                fidx = jnp.min(jnp.where(cand == m, ccand, NSUB * TK * 128),
                               axis=1, keepdims=True)
                cand = jnp.where(ccand == fidx, NEG, cand)
        val_rows.append(jnp.concatenate(vals, axis=1))

    idx_rows = []
    for rg in range(RBLK // 8):
        r0, r1 = rg * 8, rg * 8 + 8
        v4 = val_rows[rg]

        def ipass(cc, carry):
            new = []
            x4 = s_ref[r0:r1, pl.ds(pl.multiple_of(cc * (NSUB * 128),
                                                   NSUB * 128), NSUB * 128)]
            for k in range(NSUB):
                accs = carry[TK * k:TK * k + TK]
                x = x4[:, k * 128:(k + 1) * 128]
                colid = lane + (cc * NSUB + k) * 128
                new += [
                    jnp.minimum(accs[t],
                                jnp.where(x == v4[:, t:t + 1], colid, NSEL))
                    for t in range(TK)]
            return tuple(new)

        accs = lax.fori_loop(0, NCH // NSUB, ipass, (big,) * (TK * NSUB))
        idxs = []
        for t in range(TK):
            a = accs[t]
            for k in range(1, NSUB):
                a = jnp.minimum(a, accs[TK * k + t])
            idxs.append(jnp.min(a, axis=1, keepdims=True))
        idx_rows.append(jnp.concatenate(idxs, axis=1))

    idx_out[...] = jnp.concatenate(idx_rows, axis=0)


def _top4(scores):
    return pl.pallas_call(
        _top4_body,
        grid=(S // RBLK,),
        out_shape=jax.ShapeDtypeStruct((S, TK), I32),
        in_specs=[pl.BlockSpec((RBLK, NSEL), lambda i: (i, 0))],
        out_specs=pl.BlockSpec((RBLK, TK), lambda i: (i, 0)),
    )(scores)


DBLK = 512


def _post_body(hs, ao, fa, g4, fw1, fb1, fw2, fb2,
               ow1, ob1, ow2, ob2, ow3, ob3, ow4, ob4, lg, lb, out_ref):
    gathered = (g4[:, 0, :] + g4[:, 1, :] + g4[:, 2, :] + g4[:, 3, :]) \
        * np.float32(0.25)
    ffn_in = fa[...] + gathered + ao[...]
    t = jnp.maximum(ffn_in @ fw1[...] + fb1[...], 0.0)
    ffn = t @ fw2[...] + fb2[...]
    combined = hs[...] + ffn
    o = jnp.maximum(combined @ ow1[...] + ob1[...], 0.0)
    o = jnp.maximum(o @ ow2[...] + ob2[...], 0.0)
    o = jnp.maximum(o @ ow3[...] + ob3[...], 0.0)
    o = o @ ow4[...] + ob4[...]
    mu = jnp.mean(o, axis=-1, keepdims=True)
    var = jnp.mean((o - mu) ** 2, axis=-1, keepdims=True)
    out_ref[...] = (o - mu) / jnp.sqrt(var + 1e-5) * lg[...] + lb[...]


def _post(hs, attn_out, final_attn, g4, p):
    blk = lambda shape: pl.BlockSpec(shape, lambda i: (i,) + (0,) * (len(shape) - 1))
    wfull = lambda shape: pl.BlockSpec(shape, lambda i: (0,) * len(shape))
    return pl.pallas_call(
        _post_body,
        grid=(S // DBLK,),
        out_shape=jax.ShapeDtypeStruct((S, H), F32),
        in_specs=[blk((DBLK, H)), blk((DBLK, H)), blk((DBLK, H)),
                  blk((DBLK, TK, H)),
                  wfull((H, H)), wfull((1, H)), wfull((H, H)), wfull((1, H)),
                  wfull((H, 512)), wfull((1, 512)),
                  wfull((512, 2 * H)), wfull((1, 2 * H)),
                  wfull((2 * H, 2 * H)), wfull((1, 2 * H)),
                  wfull((2 * H, H)), wfull((1, H)),
                  wfull((1, H)), wfull((1, H))],
        out_specs=blk((DBLK, H)),
    )(hs, attn_out, final_attn, g4,
      p['ffn_W1'], p['ffn_b1'].reshape(1, H),
      p['ffn_W2'], p['ffn_b2'].reshape(1, H),
      p['out_W1'], p['out_b1'].reshape(1, 512),
      p['out_W2'], p['out_b2'].reshape(1, 2 * H),
      p['out_W3'], p['out_b3'].reshape(1, 2 * H),
      p['out_W4'], p['out_b4'].reshape(1, H),
      p['out_ln_g'].reshape(1, H), p['out_ln_b'].reshape(1, H))


@functools.cache
def _sc_kernels():
    mesh = plsc.VectorSubcoreMesh(core_axis_name="c", subcore_axis_name="s")
    return _make_sc_gather_wsel(mesh), _make_sc_gather_emb(mesh)


def kernel(hidden_states, attention_mask, cluster_embeddings,
           selected_token_ids, token_embeddings, params):
    p = params
    hs = hidden_states.reshape(S, H)
    ids = selected_token_ids
    sc_wsel, sc_emb = _sc_kernels()

    w2flat = p['sel_W2'].reshape(-1)
    b2p = jnp.pad(p['sel_b2'], (0, ROWLEN - VOCAB))
    wsel_flat, bsel = sc_wsel(w2flat, b2p, ids)
    wsel = wsel_flat.reshape(H, NSEL)

    query, h1 = _a0(hs, p)
    attn_out = _attn(query, cluster_embeddings, p, 'lvl_attn_')
    scores, pooled = _scores(h1, wsel, bsel.reshape(1, NSEL))
    top_idx = _top4(scores)
    key_idx = _topkeys(pooled)

    g4, keys = sc_emb(token_embeddings, ids,
                      top_idx.reshape(-1), key_idx.reshape(-1))
    final_attn = _attn(hs, keys, p, 'fin_attn_')
    out = _post(hs, attn_out, final_attn, g4.reshape(S, TK, H), p)
    return out.reshape(1, S, H)


# speedup vs baseline: 1.2381x; 1.2339x over previous
import functools

import jax
import jax.numpy as jnp
import numpy as np
from jax import lax
from jax.experimental import pallas as pl
from jax.experimental.pallas import tpu as pltpu
from jax.experimental.pallas import tpu_sc as plsc

F32 = jnp.float32
I32 = jnp.int32

S = 2048
H = 768
NSEL = 8192
KCL = 32
NH = 8
DH = 96
DHP = 128
TK = 4
TKEYS = 128
VOCAB = 50257
ROWLEN = VOCAB + 7
NEG = float(-3.0e38)

NC = 2
NS = 16
NW = NC * NS
RPW = H // NW
IPW = NSEL // NW

def _wid():
    return lax.axis_index("s") * NC + lax.axis_index("c")


def _make_sc_gather_wsel(mesh):
    return functools.partial(
        pl.kernel,
        mesh=mesh,
        out_type=(
            jax.ShapeDtypeStruct((H * NSEL,), F32),
            jax.ShapeDtypeStruct((NSEL,), F32),
        ),
        scratch_types=[
            pltpu.VMEM((ROWLEN,), F32),
            pltpu.VMEM((ROWLEN,), F32),
            pltpu.VMEM((NSEL,), I32),
            pltpu.VMEM((NSEL,), F32),
            pltpu.VMEM((NSEL,), F32),
            pltpu.SemaphoreType.DMA,
            pltpu.SemaphoreType.DMA,
            pltpu.SemaphoreType.DMA,
            pltpu.SemaphoreType.DMA,
        ],
        compiler_params=pltpu.CompilerParams(needs_layout_passes=False),
    )(_sc_gather_wsel_body)


def _sc_gather_wsel_body(w2flat, b2p, ids, wsel_out, bsel_out,
                         row0, row1, ids_v, orow0, orow1,
                         dsem0, dsem1, osem0, osem1):
    wid = _wid()
    pltpu.sync_copy(ids, ids_v)
    base = wid * RPW

    def _issue(r, buf, dsem):
        hrow = base + r
        start8 = pl.multiple_of(hrow * VOCAB // 8 * 8, 8)
        pltpu.async_copy(w2flat.at[pl.ds(start8, ROWLEN)], buf, dsem)

    def _gather_row(r, buf, obuf, osem):
        hrow = base + r
        delta = hrow * VOCAB - hrow * VOCAB // 8 * 8

        def gbody(c, carry2):
            idxs = ids_v[pl.ds(c * 16, 16)] + delta
            obuf[pl.ds(c * 16, 16)] = plsc.load_gather(buf, [idxs])
            return carry2

        lax.fori_loop(0, NSEL // 16, gbody, 0, unroll=8)
        pltpu.async_copy(obuf, wsel_out.at[pl.ds(hrow * NSEL, NSEL)], osem)

    _issue(0, row0, dsem0)
    _issue(1, row1, dsem1)

    def pair(p, carry):
        r0 = 2 * p
        for r, buf, obuf, dsem, osem in ((r0, row0, orow0, dsem0, osem0),
                                         (r0 + 1, row1, orow1, dsem1, osem1)):
            pltpu.make_async_copy(w2flat.at[pl.ds(0, ROWLEN)], buf, dsem).wait()

            @pl.when(p > 0)
            def _():
                pltpu.make_async_copy(obuf, wsel_out.at[pl.ds(0, NSEL)],
                                      osem).wait()

            _gather_row(r, buf, obuf, osem)

            @pl.when(r + 2 < RPW)
            def _():
                _issue(r + 2, buf, dsem)
        return carry

    lax.fori_loop(0, RPW // 2, pair, 0)
    pltpu.make_async_copy(orow0, wsel_out.at[pl.ds(0, NSEL)], osem0).wait()
    pltpu.make_async_copy(orow1, wsel_out.at[pl.ds(0, NSEL)], osem1).wait()

    pltpu.sync_copy(b2p, row0)

    def bbody(c, carry):
        idxs = ids_v[pl.ds(wid * IPW + c * 16, 16)]
        orow0[pl.ds(c * 16, 16)] = plsc.load_gather(row0, [idxs])
        return carry

    lax.fori_loop(0, IPW // 16, bbody, 0, unroll=4)
    pltpu.sync_copy(orow0.at[pl.ds(0, IPW)], bsel_out.at[pl.ds(wid * IPW, IPW)])


def _make_sc_gather_emb(mesh):
    return functools.partial(
        pl.kernel,
        mesh=mesh,
        out_type=(
            jax.ShapeDtypeStruct((NSEL, H), F32),
            jax.ShapeDtypeStruct((TKEYS, H), F32),
        ),
        scratch_types=[
            pltpu.VMEM((NSEL,), I32),
            pltpu.VMEM((IPW,), I32),
            pltpu.VMEM((64,), I32),
            pltpu.VMEM((64, H), F32),
            pltpu.SemaphoreType.DMA,
        ],
        compiler_params=pltpu.CompilerParams(needs_layout_passes=False),
    )(_sc_gather_emb_body)


def _sc_gather_emb_body(tok, ids, tidx, kidx, g4_out, keys_out,
                        ids_v, tidx_v, fid_v, rows_v, sem):
    wid = _wid()
    pltpu.sync_copy(ids, ids_v)
    pltpu.sync_copy(tidx.at[pl.ds(wid * IPW, IPW)], tidx_v)

    def chunk(c, carry):
        def mp(k, carry2):
            idxs = tidx_v[pl.ds(c * 64 + k * 16, 16)]
            fid_v[pl.ds(k * 16, 16)] = plsc.load_gather(ids_v, [idxs])
            return carry2

        lax.fori_loop(0, 4, mp, 0, unroll=4)
        pltpu.async_copy(tok.at[fid_v], rows_v, sem).wait()
        pltpu.sync_copy(rows_v, g4_out.at[pl.ds(wid * IPW + c * 64, 64)])
        return carry

    lax.fori_loop(0, IPW // 64, chunk, 0)

    @pl.when(wid < TKEYS // 64)
    def _():
        pltpu.sync_copy(kidx.at[pl.ds(wid * 64, 64)], tidx_v.at[pl.ds(0, 64)])

        def mp2(k, carry2):
            idxs = tidx_v[pl.ds(k * 16, 16)]
            fid_v[pl.ds(k * 16, 16)] = plsc.load_gather(ids_v, [idxs])
            return carry2

        lax.fori_loop(0, 4, mp2, 0, unroll=4)
        pltpu.async_copy(tok.at[fid_v], rows_v, sem).wait()
        pltpu.sync_copy(rows_v, keys_out.at[pl.ds(wid * 64, 64)])


def _a0_body(hs, w1, b1, w2, b2, w3, b3, g, b, sw1, sb1, q_out, h1_out):
    x = hs[...]
    h = jnp.maximum(x @ w1[...] + b1[...], 0.0)
    h = jnp.maximum(h @ w2[...] + b2[...], 0.0)
    h = h @ w3[...] + b3[...]
    mu = jnp.mean(h, axis=-1, keepdims=True)
    var = jnp.mean((h - mu) ** 2, axis=-1, keepdims=True)
    q_out[...] = (h - mu) / jnp.sqrt(var + 1e-5) * g[...] + b[...]
    h1_out[...] = jnp.maximum(x @ sw1[...] + sb1[...], 0.0)


def _a0(hs, p):
    full = lambda shape: pl.BlockSpec(shape, lambda: (0,) * len(shape))
    return pl.pallas_call(
        _a0_body,
        out_shape=(
            jax.ShapeDtypeStruct((S, H), F32),
            jax.ShapeDtypeStruct((S, H), F32),
        ),
        in_specs=[full((S, H))] + [full((H, H)), full((1, H))] * 3
        + [full((1, H)), full((1, H))] + [full((H, H)), full((1, H))],
        out_specs=(full((S, H)), full((S, H))),
    )(hs, p['lvl_W1'], p['lvl_b1'].reshape(1, H),
      p['lvl_W2'], p['lvl_b2'].reshape(1, H),
      p['lvl_W3'], p['lvl_b3'].reshape(1, H),
      p['lvl_ln_g'].reshape(1, H), p['lvl_ln_b'].reshape(1, H),
      p['sel_W1'], p['sel_b1'].reshape(1, H))


def _attn_body(q_ref, kv_ref, wq, bq, wk, bk, wv, bv, wo, bo, out_ref):
    h = pl.program_id(0)
    scale = np.float32(np.sqrt(DH))
    qh = q_ref[...] @ wq[0] + bq[0]
    kh = kv_ref[...] @ wk[0] + bk[0]
    vh = kv_ref[...] @ wv[0] + bv[0]
    s = lax.dot_general(qh, kh, (((1,), (1,)), ((), ()))) / scale
    m = jnp.max(s, axis=-1, keepdims=True)
    e = jnp.exp(s - m)
    w = e / jnp.sum(e, axis=-1, keepdims=True)
    contrib = (w @ vh) @ wo[0]

    @pl.when(h == 0)
    def _():
        out_ref[...] = contrib + bo[...]

    @pl.when(h > 0)
    def _():
        out_ref[...] += contrib


def _head_w(W):
    Wh = W.reshape(H, NH, DH).transpose(1, 0, 2)
    return jnp.pad(Wh, ((0, 0), (0, 0), (0, DHP - DH)))


def _head_b(b):
    return jnp.pad(b.reshape(NH, 1, DH), ((0, 0), (0, 0), (0, DHP - DH)))


def _head_wo(W):
    return jnp.pad(W.reshape(NH, DH, H), ((0, 0), (0, DHP - DH), (0, 0)))


def _attn(q, kv, p, pre):
    nkv = kv.shape[0]
    full = lambda shape: pl.BlockSpec(shape, lambda h: (0,) * len(shape))
    headw = lambda shape: pl.BlockSpec((1,) + shape, lambda h: (h, 0, 0))
    return pl.pallas_call(
        _attn_body,
        grid=(NH,),
        out_shape=jax.ShapeDtypeStruct((S, H), F32),
        in_specs=[full((S, H)), full((nkv, H)),
                  headw((H, DHP)), headw((1, DHP)),
                  headw((H, DHP)), headw((1, DHP)),
                  headw((H, DHP)), headw((1, DHP)),
                  headw((DHP, H)), full((1, H))],
        out_specs=full((S, H)),
    )(q, kv,
      _head_w(p[pre + 'Wq']), _head_b(p[pre + 'bq']),
      _head_w(p[pre + 'Wk']), _head_b(p[pre + 'bk']),
      _head_w(p[pre + 'Wv']), _head_b(p[pre + 'bv']),
      _head_wo(p[pre + 'Wo']), p[pre + 'bo'].reshape(1, H))


RBLK = 128
NRB = S // RBLK


def _iter_topk(v, k, width):
    cols = lax.broadcasted_iota(I32, v.shape, 1)
    lane = lax.broadcasted_iota(I32, (v.shape[0], k), 1)
    out = jnp.zeros((v.shape[0], k), I32)
    for t in range(k):
        m = jnp.max(v, axis=1, keepdims=True)
        idx = jnp.min(jnp.where(v == m, cols, width), axis=1, keepdims=True)
        out = jnp.where(lane == t, idx, out)
        if t + 1 < k:
            v = jnp.where(cols == idx, NEG, v)
    return out


NCH = NSEL // 128
NBLK = 1024


def _scores_body(h1_ref, w_ref, b_ref, s_out, p_out):
    s = h1_ref[...] @ w_ref[...] + b_ref[...]
    s_out[...] = s
    p_out[...] = jnp.sum(s, axis=0, keepdims=True) * np.float32(1.0 / S)


def _scores(h1, wsel, bsel):
    return pl.pallas_call(
        _scores_body,
        grid=(NSEL // NBLK,),
        out_shape=(
            jax.ShapeDtypeStruct((S, NSEL), F32),
            jax.ShapeDtypeStruct((1, NSEL), F32),
        ),
        in_specs=[pl.BlockSpec((S, H), lambda j: (0, 0)),
                  pl.BlockSpec((H, NBLK), lambda j: (0, j)),
                  pl.BlockSpec((1, NBLK), lambda j: (0, j))],
        out_specs=(pl.BlockSpec((S, NBLK), lambda j: (0, j)),
                   pl.BlockSpec((1, NBLK), lambda j: (0, j))),
    )(h1, wsel, bsel)


def _topkeys(pooled):
    full = lambda shape: pl.BlockSpec(shape, lambda: (0,) * len(shape))
    return pl.pallas_call(
        lambda p_ref, idx_out: idx_out.__setitem__(
            ..., _iter_topk(p_ref[...], TKEYS, NSEL)),
        out_shape=jax.ShapeDtypeStruct((1, TKEYS), I32),
        in_specs=[full((1, NSEL))],
        out_specs=full((1, TKEYS)),
    )(pooled)


def _top4_body(s_ref, idx_out):
    lane = lax.broadcasted_iota(I32, (8, 128), 1)
    neg = jnp.full((8, 128), NEG, F32)
    big = jnp.full((8, 128), NSEL, I32)
    NSUB = 4
    ccand = lax.broadcasted_iota(I32, (8, NSUB * TK * 128), 1)

    val_rows = []
    for rg in range(RBLK // 8):
        r0, r1 = rg * 8, rg * 8 + 8

        def merge(cc, carry):
            new = []
            x4 = s_ref[r0:r1, pl.ds(pl.multiple_of(cc * (NSUB * 128),
                                                   NSUB * 128), NSUB * 128)]
            for k in range(NSUB):
                t0, t1, t2, t3 = carry[TK * k:TK * k + TK]
                x = x4[:, k * 128:(k + 1) * 128]
                n0 = jnp.maximum(t0, x)
                q0 = jnp.minimum(t0, x)
                n1 = jnp.maximum(t1, q0)
                q1 = jnp.minimum(t1, q0)
                n2 = jnp.maximum(t2, q1)
                q2 = jnp.minimum(t2, q1)
                n3 = jnp.maximum(t3, q2)
                new += [n0, n1, n2, n3]
            return tuple(new)

        planes = lax.fori_loop(0, NCH // NSUB, merge, (neg,) * (TK * NSUB))
        cand = jnp.concatenate(list(planes), axis=1)
        vals = []
        for t in range(TK):
            m = jnp.max(cand, axis=1, keepdims=True)
            vals.append(m)
            if t + 1 < TK:
                fidx = jnp.min(jnp.where(cand == m, ccand, NSUB * TK * 128),
                               axis=1, keepdims=True)
                cand = jnp.where(ccand == fidx, NEG, cand)
        val_rows.append(jnp.concatenate(vals, axis=1))

    idx_rows = []
    for rg in range(RBLK // 8):
        r0, r1 = rg * 8, rg * 8 + 8
        v4 = val_rows[rg]

        def ipass(cc, carry):
            new = []
            x4 = s_ref[r0:r1, pl.ds(pl.multiple_of(cc * (NSUB * 128),
                                                   NSUB * 128), NSUB * 128)]
            for k in range(NSUB):
                accs = carry[TK * k:TK * k + TK]
                x = x4[:, k * 128:(k + 1) * 128]
                colid = lane + (cc * NSUB + k) * 128
                new += [
                    jnp.minimum(accs[t],
                                jnp.where(x == v4[:, t:t + 1], colid, NSEL))
                    for t in range(TK)]
            return tuple(new)

        accs = lax.fori_loop(0, NCH // NSUB, ipass, (big,) * (TK * NSUB))
        idxs = []
        for t in range(TK):
            a = accs[t]
            for k in range(1, NSUB):
                a = jnp.minimum(a, accs[TK * k + t])
            idxs.append(jnp.min(a, axis=1, keepdims=True))
        idx_rows.append(jnp.concatenate(idxs, axis=1))

    idx_out[...] = jnp.concatenate(idx_rows, axis=0)


def _top4_iter_body(s_ref, idx_out):
    idx_out[...] = _iter_topk(s_ref[...], TK, NSEL)


def _top4(scores):
    return pl.pallas_call(
        _top4_iter_body,
        grid=(S // RBLK,),
        out_shape=jax.ShapeDtypeStruct((S, TK), I32),
        in_specs=[pl.BlockSpec((RBLK, NSEL), lambda i: (i, 0))],
        out_specs=pl.BlockSpec((RBLK, TK), lambda i: (i, 0)),
    )(scores)


DBLK = 512


def _post_body(hs, ao, fa, g4, fw1, fb1, fw2, fb2,
               ow1, ob1, ow2, ob2, ow3, ob3, ow4, ob4, lg, lb, out_ref):
    gathered = (g4[:, 0, :] + g4[:, 1, :] + g4[:, 2, :] + g4[:, 3, :]) \
        * np.float32(0.25)
    ffn_in = fa[...] + gathered + ao[...]
    t = jnp.maximum(ffn_in @ fw1[...] + fb1[...], 0.0)
    ffn = t @ fw2[...] + fb2[...]
    combined = hs[...] + ffn
    o = jnp.maximum(combined @ ow1[...] + ob1[...], 0.0)
    o = jnp.maximum(o @ ow2[...] + ob2[...], 0.0)
    o = jnp.maximum(o @ ow3[...] + ob3[...], 0.0)
    o = o @ ow4[...] + ob4[...]
    mu = jnp.mean(o, axis=-1, keepdims=True)
    var = jnp.mean((o - mu) ** 2, axis=-1, keepdims=True)
    out_ref[...] = (o - mu) / jnp.sqrt(var + 1e-5) * lg[...] + lb[...]


def _post(hs, attn_out, final_attn, g4, p):
    blk = lambda shape: pl.BlockSpec(shape, lambda i: (i,) + (0,) * (len(shape) - 1))
    wfull = lambda shape: pl.BlockSpec(shape, lambda i: (0,) * len(shape))
    return pl.pallas_call(
        _post_body,
        grid=(S // DBLK,),
        out_shape=jax.ShapeDtypeStruct((S, H), F32),
        in_specs=[blk((DBLK, H)), blk((DBLK, H)), blk((DBLK, H)),
                  blk((DBLK, TK, H)),
                  wfull((H, H)), wfull((1, H)), wfull((H, H)), wfull((1, H)),
                  wfull((H, 512)), wfull((1, 512)),
                  wfull((512, 2 * H)), wfull((1, 2 * H)),
                  wfull((2 * H, 2 * H)), wfull((1, 2 * H)),
                  wfull((2 * H, H)), wfull((1, H)),
                  wfull((1, H)), wfull((1, H))],
        out_specs=blk((DBLK, H)),
    )(hs, attn_out, final_attn, g4,
      p['ffn_W1'], p['ffn_b1'].reshape(1, H),
      p['ffn_W2'], p['ffn_b2'].reshape(1, H),
      p['out_W1'], p['out_b1'].reshape(1, 512),
      p['out_W2'], p['out_b2'].reshape(1, 2 * H),
      p['out_W3'], p['out_b3'].reshape(1, 2 * H),
      p['out_W4'], p['out_b4'].reshape(1, H),
      p['out_ln_g'].reshape(1, H), p['out_ln_b'].reshape(1, H))


@functools.cache
def _sc_kernels():
    mesh = plsc.VectorSubcoreMesh(core_axis_name="c", subcore_axis_name="s")
    return _make_sc_gather_wsel(mesh), _make_sc_gather_emb(mesh)


def kernel(hidden_states, attention_mask, cluster_embeddings,
           selected_token_ids, token_embeddings, params):
    p = params
    hs = hidden_states.reshape(S, H)
    ids = selected_token_ids
    sc_wsel, sc_emb = _sc_kernels()

    w2flat = p['sel_W2'].reshape(-1)
    b2p = jnp.pad(p['sel_b2'], (0, ROWLEN - VOCAB))
    wsel_flat, bsel = sc_wsel(w2flat, b2p, ids)
    wsel = wsel_flat.reshape(H, NSEL)

    query, h1 = _a0(hs, p)
    attn_out = _attn(query, cluster_embeddings, p, 'lvl_attn_')
    scores, pooled = _scores(h1, wsel, bsel.reshape(1, NSEL))
    top_idx = _top4(scores)
    key_idx = _topkeys(pooled)

    g4, keys = sc_emb(token_embeddings, ids,
                      top_idx.reshape(-1), key_idx.reshape(-1))
    final_attn = _attn(hs, keys, p, 'fin_attn_')
    out = _post(hs, attn_out, final_attn, g4.reshape(S, TK, H), p)
    return out.reshape(1, S, H)


# speedup vs baseline: 2.7537x; 2.2242x over previous
import functools

import jax
import jax.numpy as jnp
import numpy as np
from jax import lax
from jax.experimental import pallas as pl
from jax.experimental.pallas import tpu as pltpu
from jax.experimental.pallas import tpu_sc as plsc

F32 = jnp.float32
I32 = jnp.int32

S = 2048
H = 768
NSEL = 8192
KCL = 32
NH = 8
DH = 96
DHP = 128
TK = 4
TKEYS = 128
VOCAB = 50257
ROWLEN = VOCAB + 7
NEG = float(-3.0e38)

NC = 2
NS = 16
NW = NC * NS
RPW = H // NW
IPW = NSEL // NW

def _wid():
    return lax.axis_index("s") * NC + lax.axis_index("c")


VPAD = 50272


def _make_sc_gather_wsel(mesh):
    return functools.partial(
        pl.kernel,
        mesh=mesh,
        out_type=(
            jax.ShapeDtypeStruct((H * NSEL,), F32),
            jax.ShapeDtypeStruct((NSEL,), F32),
        ),
        scratch_types=[
            pltpu.VMEM((VPAD,), F32),
            pltpu.VMEM((NSEL,), I32),
            pltpu.VMEM((NSEL,), F32),
        ],
        compiler_params=pltpu.CompilerParams(needs_layout_passes=False),
    )(_sc_gather_wsel_body)


def _sc_gather_wsel_body(w2p, b2p, ids, wsel_out, bsel_out, row_v, ids_v, orow_v):
    wid = _wid()
    pltpu.sync_copy(ids, ids_v)

    def row_body(r, carry):
        hrow = wid * RPW + r
        pltpu.sync_copy(w2p.at[pl.ds(hrow * VPAD, VPAD)], row_v)

        def gbody(c, carry2):
            idxs = ids_v[pl.ds(c * 16, 16)]
            orow_v[pl.ds(c * 16, 16)] = plsc.load_gather(row_v, [idxs])
            return carry2

        lax.fori_loop(0, NSEL // 16, gbody, 0, unroll=8)
        pltpu.sync_copy(orow_v, wsel_out.at[pl.ds(hrow * NSEL, NSEL)])
        return carry

    lax.fori_loop(0, RPW, row_body, 0)

    pltpu.sync_copy(b2p, row_v)

    def bbody(c, carry):
        idxs = ids_v[pl.ds(wid * IPW + c * 16, 16)]
        orow_v[pl.ds(c * 16, 16)] = plsc.load_gather(row_v, [idxs])
        return carry

    lax.fori_loop(0, IPW // 16, bbody, 0, unroll=4)
    pltpu.sync_copy(orow_v.at[pl.ds(0, IPW)], bsel_out.at[pl.ds(wid * IPW, IPW)])


def _make_sc_gather_emb(mesh):
    return functools.partial(
        pl.kernel,
        mesh=mesh,
        out_type=(
            jax.ShapeDtypeStruct((NSEL, H), F32),
            jax.ShapeDtypeStruct((TKEYS, H), F32),
        ),
        scratch_types=[
            pltpu.VMEM((NSEL,), I32),
            pltpu.VMEM((IPW,), I32),
            pltpu.VMEM((64,), I32),
            pltpu.VMEM((64, H), F32),
            pltpu.SemaphoreType.DMA,
        ],
        compiler_params=pltpu.CompilerParams(needs_layout_passes=False),
    )(_sc_gather_emb_body)


def _sc_gather_emb_body(tok, ids, tidx, kidx, g4_out, keys_out,
                        ids_v, tidx_v, fid_v, rows_v, sem):
    wid = _wid()
    pltpu.sync_copy(ids, ids_v)
    pltpu.sync_copy(tidx.at[pl.ds(wid * IPW, IPW)], tidx_v)

    def chunk(c, carry):
        def mp(k, carry2):
            idxs = tidx_v[pl.ds(c * 64 + k * 16, 16)]
            fid_v[pl.ds(k * 16, 16)] = plsc.load_gather(ids_v, [idxs])
            return carry2

        lax.fori_loop(0, 4, mp, 0, unroll=4)
        pltpu.async_copy(tok.at[fid_v], rows_v, sem).wait()
        pltpu.sync_copy(rows_v, g4_out.at[pl.ds(wid * IPW + c * 64, 64)])
        return carry

    lax.fori_loop(0, IPW // 64, chunk, 0)

    @pl.when(wid < TKEYS // 64)
    def _():
        pltpu.sync_copy(kidx.at[pl.ds(wid * 64, 64)], tidx_v.at[pl.ds(0, 64)])

        def mp2(k, carry2):
            idxs = tidx_v[pl.ds(k * 16, 16)]
            fid_v[pl.ds(k * 16, 16)] = plsc.load_gather(ids_v, [idxs])
            return carry2

        lax.fori_loop(0, 4, mp2, 0, unroll=4)
        pltpu.async_copy(tok.at[fid_v], rows_v, sem).wait()
        pltpu.sync_copy(rows_v, keys_out.at[pl.ds(wid * 64, 64)])


def _a0_body(hs, w1, b1, w2, b2, w3, b3, g, b, sw1, sb1, q_out, h1_out):
    x = hs[...]
    h = jnp.maximum(x @ w1[...] + b1[...], 0.0)
    h = jnp.maximum(h @ w2[...] + b2[...], 0.0)
    h = h @ w3[...] + b3[...]
    mu = jnp.mean(h, axis=-1, keepdims=True)
    var = jnp.mean((h - mu) ** 2, axis=-1, keepdims=True)
    q_out[...] = (h - mu) / jnp.sqrt(var + 1e-5) * g[...] + b[...]
    h1_out[...] = jnp.maximum(x @ sw1[...] + sb1[...], 0.0)


def _a0(hs, p):
    full = lambda shape: pl.BlockSpec(shape, lambda: (0,) * len(shape))
    return pl.pallas_call(
        _a0_body,
        out_shape=(
            jax.ShapeDtypeStruct((S, H), F32),
            jax.ShapeDtypeStruct((S, H), F32),
        ),
        in_specs=[full((S, H))] + [full((H, H)), full((1, H))] * 3
        + [full((1, H)), full((1, H))] + [full((H, H)), full((1, H))],
        out_specs=(full((S, H)), full((S, H))),
    )(hs, p['lvl_W1'], p['lvl_b1'].reshape(1, H),
      p['lvl_W2'], p['lvl_b2'].reshape(1, H),
      p['lvl_W3'], p['lvl_b3'].reshape(1, H),
      p['lvl_ln_g'].reshape(1, H), p['lvl_ln_b'].reshape(1, H),
      p['sel_W1'], p['sel_b1'].reshape(1, H))


def _attn_body(q_ref, kv_ref, wq, bq, wk, bk, wv, bv, wo, bo, out_ref):
    h = pl.program_id(0)
    scale = np.float32(np.sqrt(DH))
    qh = q_ref[...] @ wq[0] + bq[0]
    kh = kv_ref[...] @ wk[0] + bk[0]
    vh = kv_ref[...] @ wv[0] + bv[0]
    s = lax.dot_general(qh, kh, (((1,), (1,)), ((), ()))) / scale
    m = jnp.max(s, axis=-1, keepdims=True)
    e = jnp.exp(s - m)
    w = e / jnp.sum(e, axis=-1, keepdims=True)
    contrib = (w @ vh) @ wo[0]

    @pl.when(h == 0)
    def _():
        out_ref[...] = contrib + bo[...]

    @pl.when(h > 0)
    def _():
        out_ref[...] += contrib


def _head_w(W):
    Wh = W.reshape(H, NH, DH).transpose(1, 0, 2)
    return jnp.pad(Wh, ((0, 0), (0, 0), (0, DHP - DH)))


def _head_b(b):
    return jnp.pad(b.reshape(NH, 1, DH), ((0, 0), (0, 0), (0, DHP - DH)))


def _head_wo(W):
    return jnp.pad(W.reshape(NH, DH, H), ((0, 0), (0, DHP - DH), (0, 0)))


def _attn(q, kv, p, pre):
    nkv = kv.shape[0]
    full = lambda shape: pl.BlockSpec(shape, lambda h: (0,) * len(shape))
    headw = lambda shape: pl.BlockSpec((1,) + shape, lambda h: (h, 0, 0))
    return pl.pallas_call(
        _attn_body,
        grid=(NH,),
        out_shape=jax.ShapeDtypeStruct((S, H), F32),
        in_specs=[full((S, H)), full((nkv, H)),
                  headw((H, DHP)), headw((1, DHP)),
                  headw((H, DHP)), headw((1, DHP)),
                  headw((H, DHP)), headw((1, DHP)),
                  headw((DHP, H)), full((1, H))],
        out_specs=full((S, H)),
    )(q, kv,
      _head_w(p[pre + 'Wq']), _head_b(p[pre + 'bq']),
      _head_w(p[pre + 'Wk']), _head_b(p[pre + 'bk']),
      _head_w(p[pre + 'Wv']), _head_b(p[pre + 'bv']),
      _head_wo(p[pre + 'Wo']), p[pre + 'bo'].reshape(1, H))


RBLK = 128
NRB = S // RBLK


def _iter_topk(v, k, width):
    cols = lax.broadcasted_iota(I32, v.shape, 1)
    lane = lax.broadcasted_iota(I32, (v.shape[0], k), 1)
    out = jnp.zeros((v.shape[0], k), I32)
    for t in range(k):
        m = jnp.max(v, axis=1, keepdims=True)
        idx = jnp.min(jnp.where(v == m, cols, width), axis=1, keepdims=True)
        out = jnp.where(lane == t, idx, out)
        if t + 1 < k:
            v = jnp.where(cols == idx, NEG, v)
    return out


NCH = NSEL // 128
NBLK = 1024


def _scores_body(h1_ref, w_ref, b_ref, s_out, p_out):
    s = h1_ref[...] @ w_ref[...] + b_ref[...]
    s_out[...] = s
    p_out[...] = jnp.sum(s, axis=0, keepdims=True) * np.float32(1.0 / S)


def _scores(h1, wsel, bsel):
    return pl.pallas_call(
        _scores_body,
        grid=(NSEL // NBLK,),
        out_shape=(
            jax.ShapeDtypeStruct((S, NSEL), F32),
            jax.ShapeDtypeStruct((1, NSEL), F32),
        ),
        in_specs=[pl.BlockSpec((S, H), lambda j: (0, 0)),
                  pl.BlockSpec((H, NBLK), lambda j: (0, j)),
                  pl.BlockSpec((1, NBLK), lambda j: (0, j))],
        out_specs=(pl.BlockSpec((S, NBLK), lambda j: (0, j)),
                   pl.BlockSpec((1, NBLK), lambda j: (0, j))),
    )(h1, wsel, bsel)


def _topkeys(pooled):
    full = lambda shape: pl.BlockSpec(shape, lambda: (0,) * len(shape))
    return pl.pallas_call(
        lambda p_ref, idx_out: idx_out.__setitem__(
            ..., _iter_topk(p_ref[...], TKEYS, NSEL)),
        out_shape=jax.ShapeDtypeStruct((1, TKEYS), I32),
        in_specs=[full((1, NSEL))],
        out_specs=full((1, TKEYS)),
    )(pooled)


def _top4_body(s_ref, idx_out):
    lane = lax.broadcasted_iota(I32, (8, 128), 1)
    neg = jnp.full((8, 128), NEG, F32)
    big = jnp.full((8, 128), NSEL, I32)
    NSUB = 4
    ccand = lax.broadcasted_iota(I32, (8, NSUB * TK * 128), 1)

    val_rows = []
    for rg in range(RBLK // 8):
        r0, r1 = rg * 8, rg * 8 + 8

        def merge(cc, carry):
            new = []
            x4 = s_ref[r0:r1, pl.ds(pl.multiple_of(cc * (NSUB * 128),
                                                   NSUB * 128), NSUB * 128)]
            for k in range(NSUB):
                t0, t1, t2, t3 = carry[TK * k:TK * k + TK]
                x = x4[:, k * 128:(k + 1) * 128]
                n0 = jnp.maximum(t0, x)
                q0 = jnp.minimum(t0, x)
                n1 = jnp.maximum(t1, q0)
                q1 = jnp.minimum(t1, q0)
                n2 = jnp.maximum(t2, q1)
                q2 = jnp.minimum(t2, q1)
                n3 = jnp.maximum(t3, q2)
                new += [n0, n1, n2, n3]
            return tuple(new)

        planes = lax.fori_loop(0, NCH // NSUB, merge, (neg,) * (TK * NSUB))
        cand = jnp.concatenate(list(planes), axis=1)
        vals = []
        for t in range(TK):
            m = jnp.max(cand, axis=1, keepdims=True)
            vals.append(m)
            if t + 1 < TK:
                fidx = jnp.min(jnp.where(cand == m, ccand, NSUB * TK * 128),
                               axis=1, keepdims=True)
                cand = jnp.where(ccand == fidx, NEG, cand)
        val_rows.append(jnp.concatenate(vals, axis=1))

    idx_rows = []
    for rg in range(RBLK // 8):
        r0, r1 = rg * 8, rg * 8 + 8
        v4 = val_rows[rg]

        def ipass(cc, carry):
            new = []
            x4 = s_ref[r0:r1, pl.ds(pl.multiple_of(cc * (NSUB * 128),
                                                   NSUB * 128), NSUB * 128)]
            for k in range(NSUB):
                accs = carry[TK * k:TK * k + TK]
                x = x4[:, k * 128:(k + 1) * 128]
                colid = lane + (cc * NSUB + k) * 128
                new += [
                    jnp.minimum(accs[t],
                                jnp.where(x == v4[:, t:t + 1], colid, NSEL))
                    for t in range(TK)]
            return tuple(new)

        accs = lax.fori_loop(0, NCH // NSUB, ipass, (big,) * (TK * NSUB))
        idxs = []
        for t in range(TK):
            a = accs[t]
            for k in range(1, NSUB):
                a = jnp.minimum(a, accs[TK * k + t])
            idxs.append(jnp.min(a, axis=1, keepdims=True))
        idx_rows.append(jnp.concatenate(idxs, axis=1))

    idx_out[...] = jnp.concatenate(idx_rows, axis=0)


def _top4_iter_body(s_ref, idx_out):
    idx_out[...] = _iter_topk(s_ref[...], TK, NSEL)


def _top4(scores):
    return pl.pallas_call(
        _top4_iter_body,
        grid=(S // RBLK,),
        out_shape=jax.ShapeDtypeStruct((S, TK), I32),
        in_specs=[pl.BlockSpec((RBLK, NSEL), lambda i: (i, 0))],
        out_specs=pl.BlockSpec((RBLK, TK), lambda i: (i, 0)),
    )(scores)


DBLK = 512


def _post_body(hs, ao, fa, g4, fw1, fb1, fw2, fb2,
               ow1, ob1, ow2, ob2, ow3, ob3, ow4, ob4, lg, lb, out_ref):
    gathered = (g4[:, 0, :] + g4[:, 1, :] + g4[:, 2, :] + g4[:, 3, :]) \
        * np.float32(0.25)
    ffn_in = fa[...] + gathered + ao[...]
    t = jnp.maximum(ffn_in @ fw1[...] + fb1[...], 0.0)
    ffn = t @ fw2[...] + fb2[...]
    combined = hs[...] + ffn
    o = jnp.maximum(combined @ ow1[...] + ob1[...], 0.0)
    o = jnp.maximum(o @ ow2[...] + ob2[...], 0.0)
    o = jnp.maximum(o @ ow3[...] + ob3[...], 0.0)
    o = o @ ow4[...] + ob4[...]
    mu = jnp.mean(o, axis=-1, keepdims=True)
    var = jnp.mean((o - mu) ** 2, axis=-1, keepdims=True)
    out_ref[...] = (o - mu) / jnp.sqrt(var + 1e-5) * lg[...] + lb[...]


def _post(hs, attn_out, final_attn, g4, p):
    blk = lambda shape: pl.BlockSpec(shape, lambda i: (i,) + (0,) * (len(shape) - 1))
    wfull = lambda shape: pl.BlockSpec(shape, lambda i: (0,) * len(shape))
    return pl.pallas_call(
        _post_body,
        grid=(S // DBLK,),
        out_shape=jax.ShapeDtypeStruct((S, H), F32),
        in_specs=[blk((DBLK, H)), blk((DBLK, H)), blk((DBLK, H)),
                  blk((DBLK, TK, H)),
                  wfull((H, H)), wfull((1, H)), wfull((H, H)), wfull((1, H)),
                  wfull((H, 512)), wfull((1, 512)),
                  wfull((512, 2 * H)), wfull((1, 2 * H)),
                  wfull((2 * H, 2 * H)), wfull((1, 2 * H)),
                  wfull((2 * H, H)), wfull((1, H)),
                  wfull((1, H)), wfull((1, H))],
        out_specs=blk((DBLK, H)),
    )(hs, attn_out, final_attn, g4,
      p['ffn_W1'], p['ffn_b1'].reshape(1, H),
      p['ffn_W2'], p['ffn_b2'].reshape(1, H),
      p['out_W1'], p['out_b1'].reshape(1, 512),
      p['out_W2'], p['out_b2'].reshape(1, 2 * H),
      p['out_W3'], p['out_b3'].reshape(1, 2 * H),
      p['out_W4'], p['out_b4'].reshape(1, H),
      p['out_ln_g'].reshape(1, H), p['out_ln_b'].reshape(1, H))


@functools.cache
def _sc_kernels():
    mesh = plsc.VectorSubcoreMesh(core_axis_name="c", subcore_axis_name="s")
    return _make_sc_gather_wsel(mesh), _make_sc_gather_emb(mesh)


def kernel(hidden_states, attention_mask, cluster_embeddings,
           selected_token_ids, token_embeddings, params):
    p = params
    hs = hidden_states.reshape(S, H)
    ids = selected_token_ids
    sc_wsel, sc_emb = _sc_kernels()

    w2p = jnp.pad(p['sel_W2'], ((0, 0), (0, VPAD - VOCAB))).reshape(-1)
    b2p = jnp.pad(p['sel_b2'], (0, VPAD - VOCAB))
    wsel_flat, bsel = sc_wsel(w2p, b2p, ids)
    wsel = wsel_flat.reshape(H, NSEL)

    query, h1 = _a0(hs, p)
    attn_out = _attn(query, cluster_embeddings, p, 'lvl_attn_')
    scores, pooled = _scores(h1, wsel, bsel.reshape(1, NSEL))
    top_idx = _top4(scores)
    key_idx = _topkeys(pooled)

    g4, keys = sc_emb(token_embeddings, ids,
                      top_idx.reshape(-1), key_idx.reshape(-1))
    final_attn = _attn(hs, keys, p, 'fin_attn_')
    out = _post(hs, attn_out, final_attn, g4.reshape(S, TK, H), p)
    return out.reshape(1, S, H)


# speedup vs baseline: 2.9133x; 1.0580x over previous
import functools

import jax
import jax.numpy as jnp
import numpy as np
from jax import lax
from jax.experimental import pallas as pl
from jax.experimental.pallas import tpu as pltpu
from jax.experimental.pallas import tpu_sc as plsc

F32 = jnp.float32
I32 = jnp.int32

S = 2048
H = 768
NSEL = 8192
KCL = 32
NH = 8
DH = 96
DHP = 128
TK = 4
TKEYS = 128
VOCAB = 50257
ROWLEN = VOCAB + 7
NEG = float(-3.0e38)

NC = 2
NS = 16
NW = NC * NS
RPW = H // NW
IPW = NSEL // NW

def _wid():
    return lax.axis_index("s") * NC + lax.axis_index("c")


VPAD = 50272


def _make_sc_gather_wsel(mesh):
    return functools.partial(
        pl.kernel,
        mesh=mesh,
        out_type=(
            jax.ShapeDtypeStruct((H * NSEL,), F32),
            jax.ShapeDtypeStruct((NSEL,), F32),
        ),
        scratch_types=[
            pltpu.VMEM((VPAD,), F32),
            pltpu.VMEM((VPAD,), F32),
            pltpu.VMEM((NSEL,), I32),
            pltpu.VMEM((NSEL,), F32),
            pltpu.SemaphoreType.DMA,
            pltpu.SemaphoreType.DMA,
        ],
        compiler_params=pltpu.CompilerParams(needs_layout_passes=False),
    )(_sc_gather_wsel_body)


def _sc_gather_wsel_body(w2p, b2p, ids, wsel_out, bsel_out,
                         row0, row1, ids_v, orow_v, dsem0, dsem1):
    wid = _wid()
    pltpu.sync_copy(ids, ids_v)
    row_v = row0

    def _issue(r, buf, dsem):
        hrow = wid * RPW + r
        pltpu.async_copy(w2p.at[pl.ds(hrow * VPAD, VPAD)], buf, dsem)

    def _do_row(r, buf):
        hrow = wid * RPW + r

        def gbody(c, carry2):
            idxs = ids_v[pl.ds(c * 16, 16)]
            orow_v[pl.ds(c * 16, 16)] = plsc.load_gather(buf, [idxs])
            return carry2

        lax.fori_loop(0, NSEL // 16, gbody, 0, unroll=8)
        pltpu.sync_copy(orow_v, wsel_out.at[pl.ds(hrow * NSEL, NSEL)])

    _issue(0, row0, dsem0)
    _issue(1, row1, dsem1)

    def pair(p, carry):
        r0 = 2 * p
        for r, buf, dsem in ((r0, row0, dsem0), (r0 + 1, row1, dsem1)):
            pltpu.make_async_copy(w2p.at[pl.ds(0, VPAD)], buf, dsem).wait()
            _do_row(r, buf)

            @pl.when(r + 2 < RPW)
            def _():
                _issue(r + 2, buf, dsem)
        return carry

    lax.fori_loop(0, RPW // 2, pair, 0)

    pltpu.sync_copy(b2p, row_v)

    def bbody(c, carry):
        idxs = ids_v[pl.ds(wid * IPW + c * 16, 16)]
        orow_v[pl.ds(c * 16, 16)] = plsc.load_gather(row_v, [idxs])
        return carry

    lax.fori_loop(0, IPW // 16, bbody, 0, unroll=4)
    pltpu.sync_copy(orow_v.at[pl.ds(0, IPW)], bsel_out.at[pl.ds(wid * IPW, IPW)])


def _make_sc_gather_emb(mesh):
    return functools.partial(
        pl.kernel,
        mesh=mesh,
        out_type=(
            jax.ShapeDtypeStruct((NSEL, H), F32),
            jax.ShapeDtypeStruct((TKEYS, H), F32),
        ),
        scratch_types=[
            pltpu.VMEM((NSEL,), I32),
            pltpu.VMEM((IPW,), I32),
            pltpu.VMEM((64,), I32),
            pltpu.VMEM((64, H), F32),
            pltpu.SemaphoreType.DMA,
        ],
        compiler_params=pltpu.CompilerParams(needs_layout_passes=False),
    )(_sc_gather_emb_body)


def _sc_gather_emb_body(tok, ids, tidx, kidx, g4_out, keys_out,
                        ids_v, tidx_v, fid_v, rows_v, sem):
    wid = _wid()
    pltpu.sync_copy(ids, ids_v)
    pltpu.sync_copy(tidx.at[pl.ds(wid * IPW, IPW)], tidx_v)

    def chunk(c, carry):
        def mp(k, carry2):
            idxs = tidx_v[pl.ds(c * 64 + k * 16, 16)]
            fid_v[pl.ds(k * 16, 16)] = plsc.load_gather(ids_v, [idxs])
            return carry2

        lax.fori_loop(0, 4, mp, 0, unroll=4)
        pltpu.async_copy(tok.at[fid_v], rows_v, sem).wait()
        pltpu.sync_copy(rows_v, g4_out.at[pl.ds(wid * IPW + c * 64, 64)])
        return carry

    lax.fori_loop(0, IPW // 64, chunk, 0)

    @pl.when(wid < TKEYS // 64)
    def _():
        pltpu.sync_copy(kidx.at[pl.ds(wid * 64, 64)], tidx_v.at[pl.ds(0, 64)])

        def mp2(k, carry2):
            idxs = tidx_v[pl.ds(k * 16, 16)]
            fid_v[pl.ds(k * 16, 16)] = plsc.load_gather(ids_v, [idxs])
            return carry2

        lax.fori_loop(0, 4, mp2, 0, unroll=4)
        pltpu.async_copy(tok.at[fid_v], rows_v, sem).wait()
        pltpu.sync_copy(rows_v, keys_out.at[pl.ds(wid * 64, 64)])


def _a0_body(hs, w1, b1, w2, b2, w3, b3, g, b, sw1, sb1, q_out, h1_out):
    x = hs[...]
    h = jnp.maximum(x @ w1[...] + b1[...], 0.0)
    h = jnp.maximum(h @ w2[...] + b2[...], 0.0)
    h = h @ w3[...] + b3[...]
    mu = jnp.mean(h, axis=-1, keepdims=True)
    var = jnp.mean((h - mu) ** 2, axis=-1, keepdims=True)
    q_out[...] = (h - mu) / jnp.sqrt(var + 1e-5) * g[...] + b[...]
    h1_out[...] = jnp.maximum(x @ sw1[...] + sb1[...], 0.0)


def _a0(hs, p):
    full = lambda shape: pl.BlockSpec(shape, lambda: (0,) * len(shape))
    return pl.pallas_call(
        _a0_body,
        out_shape=(
            jax.ShapeDtypeStruct((S, H), F32),
            jax.ShapeDtypeStruct((S, H), F32),
        ),
        in_specs=[full((S, H))] + [full((H, H)), full((1, H))] * 3
        + [full((1, H)), full((1, H))] + [full((H, H)), full((1, H))],
        out_specs=(full((S, H)), full((S, H))),
    )(hs, p['lvl_W1'], p['lvl_b1'].reshape(1, H),
      p['lvl_W2'], p['lvl_b2'].reshape(1, H),
      p['lvl_W3'], p['lvl_b3'].reshape(1, H),
      p['lvl_ln_g'].reshape(1, H), p['lvl_ln_b'].reshape(1, H),
      p['sel_W1'], p['sel_b1'].reshape(1, H))


def _attn_body(q_ref, kv_ref, wq, bq, wk, bk, wv, bv, wo, bo, out_ref):
    h = pl.program_id(0)
    scale = np.float32(np.sqrt(DH))
    qh = q_ref[...] @ wq[0] + bq[0]
    kh = kv_ref[...] @ wk[0] + bk[0]
    vh = kv_ref[...] @ wv[0] + bv[0]
    s = lax.dot_general(qh, kh, (((1,), (1,)), ((), ()))) / scale
    m = jnp.max(s, axis=-1, keepdims=True)
    e = jnp.exp(s - m)
    w = e / jnp.sum(e, axis=-1, keepdims=True)
    contrib = (w @ vh) @ wo[0]

    @pl.when(h == 0)
    def _():
        out_ref[...] = contrib + bo[...]

    @pl.when(h > 0)
    def _():
        out_ref[...] += contrib


def _head_w(W):
    Wh = W.reshape(H, NH, DH).transpose(1, 0, 2)
    return jnp.pad(Wh, ((0, 0), (0, 0), (0, DHP - DH)))


def _head_b(b):
    return jnp.pad(b.reshape(NH, 1, DH), ((0, 0), (0, 0), (0, DHP - DH)))


def _head_wo(W):
    return jnp.pad(W.reshape(NH, DH, H), ((0, 0), (0, DHP - DH), (0, 0)))


def _attn(q, kv, p, pre):
    nkv = kv.shape[0]
    full = lambda shape: pl.BlockSpec(shape, lambda h: (0,) * len(shape))
    headw = lambda shape: pl.BlockSpec((1,) + shape, lambda h: (h, 0, 0))
    return pl.pallas_call(
        _attn_body,
        grid=(NH,),
        out_shape=jax.ShapeDtypeStruct((S, H), F32),
        in_specs=[full((S, H)), full((nkv, H)),
                  headw((H, DHP)), headw((1, DHP)),
                  headw((H, DHP)), headw((1, DHP)),
                  headw((H, DHP)), headw((1, DHP)),
                  headw((DHP, H)), full((1, H))],
        out_specs=full((S, H)),
    )(q, kv,
      _head_w(p[pre + 'Wq']), _head_b(p[pre + 'bq']),
      _head_w(p[pre + 'Wk']), _head_b(p[pre + 'bk']),
      _head_w(p[pre + 'Wv']), _head_b(p[pre + 'bv']),
      _head_wo(p[pre + 'Wo']), p[pre + 'bo'].reshape(1, H))


RBLK = 128
NRB = S // RBLK


def _iter_topk(v, k, width):
    cols = lax.broadcasted_iota(I32, v.shape, 1)
    lane = lax.broadcasted_iota(I32, (v.shape[0], k), 1)
    out = jnp.zeros((v.shape[0], k), I32)
    for t in range(k):
        m = jnp.max(v, axis=1, keepdims=True)
        idx = jnp.min(jnp.where(v == m, cols, width), axis=1, keepdims=True)
        out = jnp.where(lane == t, idx, out)
        if t + 1 < k:
            v = jnp.where(cols == idx, NEG, v)
    return out


NCH = NSEL // 128
NBLK = 1024


def _scores_body(h1_ref, w_ref, b_ref, s_out, p_out):
    s = h1_ref[...] @ w_ref[...] + b_ref[...]
    s_out[...] = s
    p_out[...] = jnp.sum(s, axis=0, keepdims=True) * np.float32(1.0 / S)


def _scores(h1, wsel, bsel):
    return pl.pallas_call(
        _scores_body,
        grid=(NSEL // NBLK,),
        out_shape=(
            jax.ShapeDtypeStruct((S, NSEL), F32),
            jax.ShapeDtypeStruct((1, NSEL), F32),
        ),
        in_specs=[pl.BlockSpec((S, H), lambda j: (0, 0)),
                  pl.BlockSpec((H, NBLK), lambda j: (0, j)),
                  pl.BlockSpec((1, NBLK), lambda j: (0, j))],
        out_specs=(pl.BlockSpec((S, NBLK), lambda j: (0, j)),
                   pl.BlockSpec((1, NBLK), lambda j: (0, j))),
    )(h1, wsel, bsel)


def _topkeys(pooled):
    full = lambda shape: pl.BlockSpec(shape, lambda: (0,) * len(shape))
    return pl.pallas_call(
        lambda p_ref, idx_out: idx_out.__setitem__(
            ..., _iter_topk(p_ref[...], TKEYS, NSEL)),
        out_shape=jax.ShapeDtypeStruct((1, TKEYS), I32),
        in_specs=[full((1, NSEL))],
        out_specs=full((1, TKEYS)),
    )(pooled)


def _top4_body(s_ref, idx_out):
    lane = lax.broadcasted_iota(I32, (8, 128), 1)
    neg = jnp.full((8, 128), NEG, F32)
    big = jnp.full((8, 128), NSEL, I32)
    NSUB = 4
    ccand = lax.broadcasted_iota(I32, (8, NSUB * TK * 128), 1)

    val_rows = []
    for rg in range(RBLK // 8):
        r0, r1 = rg * 8, rg * 8 + 8

        def merge(cc, carry):
            new = []
            x4 = s_ref[r0:r1, pl.ds(pl.multiple_of(cc * (NSUB * 128),
                                                   NSUB * 128), NSUB * 128)]
            for k in range(NSUB):
                t0, t1, t2, t3 = carry[TK * k:TK * k + TK]
                x = x4[:, k * 128:(k + 1) * 128]
                n0 = jnp.maximum(t0, x)
                q0 = jnp.minimum(t0, x)
                n1 = jnp.maximum(t1, q0)
                q1 = jnp.minimum(t1, q0)
                n2 = jnp.maximum(t2, q1)
                q2 = jnp.minimum(t2, q1)
                n3 = jnp.maximum(t3, q2)
                new += [n0, n1, n2, n3]
            return tuple(new)

        planes = lax.fori_loop(0, NCH // NSUB, merge, (neg,) * (TK * NSUB))
        cand = jnp.concatenate(list(planes), axis=1)
        vals = []
        for t in range(TK):
            m = jnp.max(cand, axis=1, keepdims=True)
            vals.append(m)
            if t + 1 < TK:
                fidx = jnp.min(jnp.where(cand == m, ccand, NSUB * TK * 128),
                               axis=1, keepdims=True)
                cand = jnp.where(ccand == fidx, NEG, cand)
        val_rows.append(jnp.concatenate(vals, axis=1))

    idx_rows = []
    for rg in range(RBLK // 8):
        r0, r1 = rg * 8, rg * 8 + 8
        v4 = val_rows[rg]

        def ipass(cc, carry):
            new = []
            x4 = s_ref[r0:r1, pl.ds(pl.multiple_of(cc * (NSUB * 128),
                                                   NSUB * 128), NSUB * 128)]
            for k in range(NSUB):
                accs = carry[TK * k:TK * k + TK]
                x = x4[:, k * 128:(k + 1) * 128]
                colid = lane + (cc * NSUB + k) * 128
                new += [
                    jnp.minimum(accs[t],
                                jnp.where(x == v4[:, t:t + 1], colid, NSEL))
                    for t in range(TK)]
            return tuple(new)

        accs = lax.fori_loop(0, NCH // NSUB, ipass, (big,) * (TK * NSUB))
        idxs = []
        for t in range(TK):
            a = accs[t]
            for k in range(1, NSUB):
                a = jnp.minimum(a, accs[TK * k + t])
            idxs.append(jnp.min(a, axis=1, keepdims=True))
        idx_rows.append(jnp.concatenate(idxs, axis=1))

    idx_out[...] = jnp.concatenate(idx_rows, axis=0)


def _top4_iter_body(s_ref, idx_out):
    idx_out[...] = _iter_topk(s_ref[...], TK, NSEL)


def _top4(scores):
    return pl.pallas_call(
        _top4_iter_body,
        grid=(S // RBLK,),
        out_shape=jax.ShapeDtypeStruct((S, TK), I32),
        in_specs=[pl.BlockSpec((RBLK, NSEL), lambda i: (i, 0))],
        out_specs=pl.BlockSpec((RBLK, TK), lambda i: (i, 0)),
    )(scores)


DBLK = 512


def _post_body(hs, ao, fa, g4, fw1, fb1, fw2, fb2,
               ow1, ob1, ow2, ob2, ow3, ob3, ow4, ob4, lg, lb, out_ref):
    gathered = (g4[:, 0, :] + g4[:, 1, :] + g4[:, 2, :] + g4[:, 3, :]) \
        * np.float32(0.25)
    ffn_in = fa[...] + gathered + ao[...]
    t = jnp.maximum(ffn_in @ fw1[...] + fb1[...], 0.0)
    ffn = t @ fw2[...] + fb2[...]
    combined = hs[...] + ffn
    o = jnp.maximum(combined @ ow1[...] + ob1[...], 0.0)
    o = jnp.maximum(o @ ow2[...] + ob2[...], 0.0)
    o = jnp.maximum(o @ ow3[...] + ob3[...], 0.0)
    o = o @ ow4[...] + ob4[...]
    mu = jnp.mean(o, axis=-1, keepdims=True)
    var = jnp.mean((o - mu) ** 2, axis=-1, keepdims=True)
    out_ref[...] = (o - mu) / jnp.sqrt(var + 1e-5) * lg[...] + lb[...]


def _post(hs, attn_out, final_attn, g4, p):
    blk = lambda shape: pl.BlockSpec(shape, lambda i: (i,) + (0,) * (len(shape) - 1))
    wfull = lambda shape: pl.BlockSpec(shape, lambda i: (0,) * len(shape))
    return pl.pallas_call(
        _post_body,
        grid=(S // DBLK,),
        out_shape=jax.ShapeDtypeStruct((S, H), F32),
        in_specs=[blk((DBLK, H)), blk((DBLK, H)), blk((DBLK, H)),
                  blk((DBLK, TK, H)),
                  wfull((H, H)), wfull((1, H)), wfull((H, H)), wfull((1, H)),
                  wfull((H, 512)), wfull((1, 512)),
                  wfull((512, 2 * H)), wfull((1, 2 * H)),
                  wfull((2 * H, 2 * H)), wfull((1, 2 * H)),
                  wfull((2 * H, H)), wfull((1, H)),
                  wfull((1, H)), wfull((1, H))],
        out_specs=blk((DBLK, H)),
    )(hs, attn_out, final_attn, g4,
      p['ffn_W1'], p['ffn_b1'].reshape(1, H),
      p['ffn_W2'], p['ffn_b2'].reshape(1, H),
      p['out_W1'], p['out_b1'].reshape(1, 512),
      p['out_W2'], p['out_b2'].reshape(1, 2 * H),
      p['out_W3'], p['out_b3'].reshape(1, 2 * H),
      p['out_W4'], p['out_b4'].reshape(1, H),
      p['out_ln_g'].reshape(1, H), p['out_ln_b'].reshape(1, H))


@functools.cache
def _sc_kernels():
    mesh = plsc.VectorSubcoreMesh(core_axis_name="c", subcore_axis_name="s")
    return _make_sc_gather_wsel(mesh), _make_sc_gather_emb(mesh)


def kernel(hidden_states, attention_mask, cluster_embeddings,
           selected_token_ids, token_embeddings, params):
    p = params
    hs = hidden_states.reshape(S, H)
    ids = selected_token_ids
    sc_wsel, sc_emb = _sc_kernels()

    w2p = jnp.pad(p['sel_W2'], ((0, 0), (0, VPAD - VOCAB))).reshape(-1)
    b2p = jnp.pad(p['sel_b2'], (0, VPAD - VOCAB))
    wsel_flat, bsel = sc_wsel(w2p, b2p, ids)
    wsel = wsel_flat.reshape(H, NSEL)

    query, h1 = _a0(hs, p)
    attn_out = _attn(query, cluster_embeddings, p, 'lvl_attn_')
    scores, pooled = _scores(h1, wsel, bsel.reshape(1, NSEL))
    top_idx = _top4(scores)
    key_idx = _topkeys(pooled)

    g4, keys = sc_emb(token_embeddings, ids,
                      top_idx.reshape(-1), key_idx.reshape(-1))
    final_attn = _attn(hs, keys, p, 'fin_attn_')
    out = _post(hs, attn_out, final_attn, g4.reshape(S, TK, H), p)
    return out.reshape(1, S, H)
